# baseline (speedup 1.0000x reference)
import jax
import jax.numpy as jnp
from jax import lax
from jax.experimental import pallas as pl
from jax.experimental.pallas import tpu as pltpu
from jax.experimental.pallas import tpu_sc as plsc

VOCAB = 1000000
EMBED_DIM = 16
TAG_SIZE = 2
BATCH = 16384
HIST = 50

_INFO = plsc.get_sparse_core_info()
_NC = _INFO.num_cores
_NS = _INFO.num_subcores
_NW = _NC * _NS
_BPW = BATCH // _NW
_SUB = 128
_NSUB = _BPW // _SUB
_GROUPS = _BPW // 16
_TPAD = 8

_VBLK = 65536
_VGRID = -(-VOCAB // _VBLK)


def _proj_body(embt_ref, w_ref, b_ref, out0_ref, out1_ref):
    e = embt_ref[...]
    prod = jax.lax.dot_general(
        w_ref[...], e, (((1,), (0,)), ((), ())),
        preferred_element_type=jnp.float32)
    prod = prod + b_ref[...]
    out0_ref[...] = prod[0]
    out1_ref[...] = prod[1]


@jax.jit
def _tc_proj(embt, W, b2d):
    return pl.pallas_call(
        _proj_body,
        grid=(_VGRID,),
        in_specs=[
            pl.BlockSpec((EMBED_DIM, _VBLK), lambda i: (0, i)),
            pl.BlockSpec((TAG_SIZE, EMBED_DIM), lambda i: (0, 0)),
            pl.BlockSpec((TAG_SIZE, 1), lambda i: (0, 0)),
        ],
        out_specs=[
            pl.BlockSpec((_VBLK,), lambda i: (i,)),
            pl.BlockSpec((_VBLK,), lambda i: (i,)),
        ],
        out_shape=[
            jax.ShapeDtypeStruct((VOCAB,), jnp.float32),
            jax.ShapeDtypeStruct((VOCAB,), jnp.float32),
        ],
    )(embt, W, b2d)


_PCHUNK = 8000
_PGROUPS = _PCHUNK // 16
_NPCHUNK = VOCAB // _PCHUNK
_PROWS = VOCAB // 4


def _pack_body(p0_hbm, p1_hbm, out_hbm, a_v, b_v, rows_v):
    wid = lax.axis_index("s") * _NC + lax.axis_index("c")
    lanes = lax.iota(jnp.int32, 16)

    for c in range(4):
        k = wid + _NW * c

        @pl.when(k < _NPCHUNK)
        def _():
            vbase = k * _PCHUNK
            pltpu.sync_copy(p0_hbm.at[pl.ds(vbase, _PCHUNK)], a_v)
            pltpu.sync_copy(p1_hbm.at[pl.ds(vbase, _PCHUNK)], b_v)

            def weave(g, carry):
                vl = g * 16 + lanes
                r = lax.shift_right_logical(vl, 2)
                c0 = lax.shift_left(jnp.bitwise_and(vl, 3), 1)
                sl = pl.ds(g * 16, 16)
                plsc.store_scatter(rows_v, [r, c0], a_v[sl])
                plsc.store_scatter(rows_v, [r, c0 + 1], b_v[sl])
                return carry

            lax.fori_loop(0, _PGROUPS, weave, 0)
            pltpu.sync_copy(
                rows_v, out_hbm.at[pl.ds(k * (_PCHUNK // 4), _PCHUNK // 4), :])


@jax.jit
def _sc_pack(p0, p1):
    mesh = plsc.VectorSubcoreMesh(core_axis_name="c", subcore_axis_name="s")
    return pl.kernel(
        _pack_body,
        out_type=jax.ShapeDtypeStruct((_PROWS, _TPAD), jnp.float32),
        mesh=mesh,
        scratch_types=[
            pltpu.VMEM((_PCHUNK,), jnp.float32),
            pltpu.VMEM((_PCHUNK,), jnp.float32),
            pltpu.VMEM((_PCHUNK // 4, _TPAD), jnp.float32),
        ],
        compiler_params=pltpu.CompilerParams(
            needs_layout_passes=False, use_tc_tiling_on_sc=False),
    )(p0, p1)


def _body(xt_hbm, proj_hbm, out_hbm, idx_v, sidx_a, sidx_b, rows_a, rows_b,
          out_v, sem0, sem1):
    wid = lax.axis_index("s") * _NC + lax.axis_index("c")
    b0 = wid * _BPW

    lanes = lax.iota(jnp.int32, 16)

    pltpu.sync_copy(xt_hbm.at[:, pl.ds(b0, _BPW)], idx_v)

    bufs = (rows_a, rows_b)
    sbufs = (sidx_a, sidx_b)
    sems = (sem0, sem1)

    def shift(l):
        sidx = sbufs[l % 2]

        def sgrp(g, carry):
            sl = pl.ds(g * 16, 16)
            sidx[sl] = lax.shift_right_logical(idx_v[l, sl], 2)
            return carry

        lax.fori_loop(0, _GROUPS, sgrp, 0)

    def fire(l):
        buf = bufs[l % 2]
        sidx = sbufs[l % 2]
        sem = sems[l % 2]
        handles = []
        for j in range(_NSUB):
            handles.append(pltpu.async_copy(
                proj_hbm.at[sidx.at[pl.ds(j * _SUB, _SUB)]],
                buf.at[pl.ds(j * _SUB, _SUB), :],
                sem,
            ))
        return handles

    shift(0)
    pending = fire(0)
    for l in range(HIST):
        for h in pending:
            h.wait()
        if l + 1 < HIST:
            shift(l + 1)
            pending = fire(l + 1)
        rows = bufs[l % 2]

        def move(g, carry):
            row_ids = g * 16 + lanes
            sl = pl.ds(g * 16, 16)
            c0 = lax.shift_left(jnp.bitwise_and(idx_v[l, sl], 3), 1)
            out_v[l, 0, sl] = plsc.load_gather(rows, [row_ids, c0])
            out_v[l, 1, sl] = plsc.load_gather(rows, [row_ids, c0 + 1])
            return carry

        lax.fori_loop(0, _GROUPS, move, 0)

    pltpu.sync_copy(out_v, out_hbm.at[:, :, pl.ds(b0, _BPW)])


@jax.jit
def _run(xt, proj):
    mesh = plsc.VectorSubcoreMesh(core_axis_name="c", subcore_axis_name="s")
    return pl.kernel(
        _body,
        out_type=jax.ShapeDtypeStruct((HIST, TAG_SIZE, BATCH), jnp.float32),
        mesh=mesh,
        scratch_types=[
            pltpu.VMEM((HIST, _BPW), jnp.int32),
            pltpu.VMEM((_BPW,), jnp.int32),
            pltpu.VMEM((_BPW,), jnp.int32),
            pltpu.VMEM((_BPW, _TPAD), jnp.float32),
            pltpu.VMEM((_BPW, _TPAD), jnp.float32),
            pltpu.VMEM((HIST, TAG_SIZE, _BPW), jnp.float32),
            pltpu.SemaphoreType.DMA,
            pltpu.SemaphoreType.DMA,
        ],
        compiler_params=pltpu.CompilerParams(
            needs_layout_passes=False, use_tc_tiling_on_sc=False),
    )(xt, proj)


def kernel(x, emb, W, b):
    xt = jnp.swapaxes(x, 0, 1).astype(jnp.int32)
    embt = jnp.swapaxes(emb, 0, 1)
    b2d = b.astype(jnp.float32).reshape(TAG_SIZE, 1)
    p0, p1 = _tc_proj(embt, W.astype(jnp.float32), b2d)
    proj = _sc_pack(p0, p1)
    out_t = _run(xt, proj)
    return jnp.transpose(out_t, (2, 0, 1))


# speedup vs baseline: 1.2704x; 1.2704x over previous
import jax
import jax.numpy as jnp
from jax import lax
from jax.experimental import pallas as pl
from jax.experimental.pallas import tpu as pltpu
from jax.experimental.pallas import tpu_sc as plsc

VOCAB = 1000000
EMBED_DIM = 16
TAG_SIZE = 2
BATCH = 16384
HIST = 50

_INFO = plsc.get_sparse_core_info()
_NC = _INFO.num_cores
_NS = _INFO.num_subcores
_NW = _NC * _NS
_BPW = BATCH // _NW
_SUB = 128
_NSUB = _BPW // _SUB
_GROUPS = _BPW // 16
_TPAD = 8

_VBLK = 65536
_VGRID = -(-VOCAB // _VBLK)


def _proj_body(embt_ref, w_ref, b_ref, out0_ref, out1_ref):
    e = embt_ref[...]
    prod = jax.lax.dot_general(
        w_ref[...], e, (((1,), (0,)), ((), ())),
        preferred_element_type=jnp.float32)
    prod = prod + b_ref[...]
    out0_ref[...] = prod[0]
    out1_ref[...] = prod[1]


@jax.jit
def _tc_proj(embt, W, b2d):
    return pl.pallas_call(
        _proj_body,
        grid=(_VGRID,),
        in_specs=[
            pl.BlockSpec((EMBED_DIM, _VBLK), lambda i: (0, i)),
            pl.BlockSpec((TAG_SIZE, EMBED_DIM), lambda i: (0, 0)),
            pl.BlockSpec((TAG_SIZE, 1), lambda i: (0, 0)),
        ],
        out_specs=[
            pl.BlockSpec((_VBLK,), lambda i: (i,)),
            pl.BlockSpec((_VBLK,), lambda i: (i,)),
        ],
        out_shape=[
            jax.ShapeDtypeStruct((VOCAB,), jnp.float32),
            jax.ShapeDtypeStruct((VOCAB,), jnp.float32),
        ],
    )(embt, W, b2d)


_PCHUNK = 8000
_PGROUPS = _PCHUNK // 16
_NPCHUNK = VOCAB // _PCHUNK
_PROWS = VOCAB // 4


def _pack_body(p0_hbm, p1_hbm, out_hbm, a_v, b_v, rows_v):
    wid = lax.axis_index("s") * _NC + lax.axis_index("c")
    lanes = lax.iota(jnp.int32, 16)

    for c in range(4):
        k = wid + _NW * c

        @pl.when(k < _NPCHUNK)
        def _():
            vbase = k * _PCHUNK
            pltpu.sync_copy(p0_hbm.at[pl.ds(vbase, _PCHUNK)], a_v)
            pltpu.sync_copy(p1_hbm.at[pl.ds(vbase, _PCHUNK)], b_v)

            def weave(g, carry):
                vl = g * 16 + lanes
                r = lax.shift_right_logical(vl, 2)
                c0 = lax.shift_left(jnp.bitwise_and(vl, 3), 1)
                sl = pl.ds(g * 16, 16)
                plsc.store_scatter(rows_v, [r, c0], a_v[sl])
                plsc.store_scatter(rows_v, [r, c0 + 1], b_v[sl])
                return carry

            lax.fori_loop(0, _PGROUPS, weave, 0)
            pltpu.sync_copy(
                rows_v, out_hbm.at[pl.ds(k * (_PCHUNK // 4), _PCHUNK // 4), :])


@jax.jit
def _sc_pack(p0, p1):
    mesh = plsc.VectorSubcoreMesh(core_axis_name="c", subcore_axis_name="s")
    return pl.kernel(
        _pack_body,
        out_type=jax.ShapeDtypeStruct((_PROWS, _TPAD), jnp.float32),
        mesh=mesh,
        scratch_types=[
            pltpu.VMEM((_PCHUNK,), jnp.float32),
            pltpu.VMEM((_PCHUNK,), jnp.float32),
            pltpu.VMEM((_PCHUNK // 4, _TPAD), jnp.float32),
        ],
        compiler_params=pltpu.CompilerParams(
            needs_layout_passes=False, use_tc_tiling_on_sc=False),
    )(p0, p1)


_PAR = 4


def _body(xt_hbm, proj_hbm, out_hbm, idx_v,
          sidx_a, sidx_b, sidx_c, sidx_d,
          rows_a, rows_b, rows_c, rows_d,
          out_v, sem0, sem1, sem2, sem3):
    wid = lax.axis_index("s") * _NC + lax.axis_index("c")
    b0 = wid * _BPW

    lanes = lax.iota(jnp.int32, 16)

    pltpu.sync_copy(xt_hbm.at[:, pl.ds(b0, _BPW)], idx_v)

    bufs = (rows_a, rows_b, rows_c, rows_d)
    sbufs = (sidx_a, sidx_b, sidx_c, sidx_d)
    sems = (sem0, sem1, sem2, sem3)

    def shift(l):
        sidx = sbufs[l % _PAR]

        def sgrp(g, carry):
            sl = pl.ds(g * 16, 16)
            sidx[sl] = lax.shift_right_logical(idx_v[l, sl], 2)
            return carry

        lax.fori_loop(0, _GROUPS, sgrp, 0)

    def fire(l):
        buf = bufs[l % _PAR]
        sidx = sbufs[l % _PAR]
        sem = sems[l % _PAR]
        handles = []
        for j in range(_NSUB):
            handles.append(pltpu.async_copy(
                proj_hbm.at[sidx.at[pl.ds(j * _SUB, _SUB)]],
                buf.at[pl.ds(j * _SUB, _SUB), :],
                sem,
            ))
        return handles

    pending = {}
    for l in range(_PAR - 1):
        shift(l)
        pending[l] = fire(l)
    for l in range(HIST):
        for h in pending.pop(l):
            h.wait()
        if l + _PAR - 1 < HIST:
            shift(l + _PAR - 1)
            pending[l + _PAR - 1] = fire(l + _PAR - 1)
        rows = bufs[l % _PAR]

        def move(g, carry):
            row_ids = g * 16 + lanes
            sl = pl.ds(g * 16, 16)
            c0 = lax.shift_left(jnp.bitwise_and(idx_v[l, sl], 3), 1)
            out_v[l, 0, sl] = plsc.load_gather(rows, [row_ids, c0])
            out_v[l, 1, sl] = plsc.load_gather(rows, [row_ids, c0 + 1])
            return carry

        lax.fori_loop(0, _GROUPS, move, 0)

    pltpu.sync_copy(out_v, out_hbm.at[:, :, pl.ds(b0, _BPW)])


@jax.jit
def _run(xt, proj):
    mesh = plsc.VectorSubcoreMesh(core_axis_name="c", subcore_axis_name="s")
    return pl.kernel(
        _body,
        out_type=jax.ShapeDtypeStruct((HIST, TAG_SIZE, BATCH), jnp.float32),
        mesh=mesh,
        scratch_types=[
            pltpu.VMEM((HIST, _BPW), jnp.int32),
            pltpu.VMEM((_BPW,), jnp.int32),
            pltpu.VMEM((_BPW,), jnp.int32),
            pltpu.VMEM((_BPW,), jnp.int32),
            pltpu.VMEM((_BPW,), jnp.int32),
            pltpu.VMEM((_BPW, _TPAD), jnp.float32),
            pltpu.VMEM((_BPW, _TPAD), jnp.float32),
            pltpu.VMEM((_BPW, _TPAD), jnp.float32),
            pltpu.VMEM((_BPW, _TPAD), jnp.float32),
            pltpu.VMEM((HIST, TAG_SIZE, _BPW), jnp.float32),
            pltpu.SemaphoreType.DMA,
            pltpu.SemaphoreType.DMA,
            pltpu.SemaphoreType.DMA,
            pltpu.SemaphoreType.DMA,
        ],
        compiler_params=pltpu.CompilerParams(
            needs_layout_passes=False, use_tc_tiling_on_sc=False),
    )(xt, proj)


def kernel(x, emb, W, b):
    xt = jnp.swapaxes(x, 0, 1).astype(jnp.int32)
    embt = jnp.swapaxes(emb, 0, 1)
    b2d = b.astype(jnp.float32).reshape(TAG_SIZE, 1)
    p0, p1 = _tc_proj(embt, W.astype(jnp.float32), b2d)
    proj = _sc_pack(p0, p1)
    out_t = _run(xt, proj)
    return jnp.transpose(out_t, (2, 0, 1))


# speedup vs baseline: 1.2711x; 1.0005x over previous
import jax
import jax.numpy as jnp
from jax import lax
from jax.experimental import pallas as pl
from jax.experimental.pallas import tpu as pltpu
from jax.experimental.pallas import tpu_sc as plsc

VOCAB = 1000000
EMBED_DIM = 16
TAG_SIZE = 2
BATCH = 16384
HIST = 50

_INFO = plsc.get_sparse_core_info()
_NC = _INFO.num_cores
_NS = _INFO.num_subcores
_NW = _NC * _NS
_BPW = BATCH // _NW
_SUB = 128
_NSUB = _BPW // _SUB
_GROUPS = _BPW // 16
_TPAD = 8

_VBLK = 65536
_VGRID = -(-VOCAB // _VBLK)


def _proj_body(embt_ref, w_ref, b_ref, out0_ref, out1_ref):
    e = embt_ref[...]
    prod = jax.lax.dot_general(
        w_ref[...], e, (((1,), (0,)), ((), ())),
        preferred_element_type=jnp.float32)
    prod = prod + b_ref[...]
    out0_ref[...] = prod[0]
    out1_ref[...] = prod[1]


@jax.jit
def _tc_proj(embt, W, b2d):
    return pl.pallas_call(
        _proj_body,
        grid=(_VGRID,),
        in_specs=[
            pl.BlockSpec((EMBED_DIM, _VBLK), lambda i: (0, i)),
            pl.BlockSpec((TAG_SIZE, EMBED_DIM), lambda i: (0, 0)),
            pl.BlockSpec((TAG_SIZE, 1), lambda i: (0, 0)),
        ],
        out_specs=[
            pl.BlockSpec((_VBLK,), lambda i: (i,)),
            pl.BlockSpec((_VBLK,), lambda i: (i,)),
        ],
        out_shape=[
            jax.ShapeDtypeStruct((VOCAB,), jnp.float32),
            jax.ShapeDtypeStruct((VOCAB,), jnp.float32),
        ],
    )(embt, W, b2d)


_PCHUNK = 8000
_PGROUPS = _PCHUNK // 16
_NPCHUNK = VOCAB // _PCHUNK
_PROWS = VOCAB // 4


def _pack_body(p0_hbm, p1_hbm, out_hbm, a_v, b_v, rows_v):
    wid = lax.axis_index("s") * _NC + lax.axis_index("c")
    lanes = lax.iota(jnp.int32, 16)

    for c in range(4):
        k = wid + _NW * c

        @pl.when(k < _NPCHUNK)
        def _():
            vbase = k * _PCHUNK
            pltpu.sync_copy(p0_hbm.at[pl.ds(vbase, _PCHUNK)], a_v)
            pltpu.sync_copy(p1_hbm.at[pl.ds(vbase, _PCHUNK)], b_v)

            def weave(g, carry):
                vl = g * 16 + lanes
                r = lax.shift_right_logical(vl, 2)
                c0 = lax.shift_left(jnp.bitwise_and(vl, 3), 1)
                sl = pl.ds(g * 16, 16)
                plsc.store_scatter(rows_v, [r, c0], a_v[sl])
                plsc.store_scatter(rows_v, [r, c0 + 1], b_v[sl])
                return carry

            lax.fori_loop(0, _PGROUPS, weave, 0)
            pltpu.sync_copy(
                rows_v, out_hbm.at[pl.ds(k * (_PCHUNK // 4), _PCHUNK // 4), :])


@jax.jit
def _sc_pack(p0, p1):
    mesh = plsc.VectorSubcoreMesh(core_axis_name="c", subcore_axis_name="s")
    return pl.kernel(
        _pack_body,
        out_type=jax.ShapeDtypeStruct((_PROWS, _TPAD), jnp.float32),
        mesh=mesh,
        scratch_types=[
            pltpu.VMEM((_PCHUNK,), jnp.float32),
            pltpu.VMEM((_PCHUNK,), jnp.float32),
            pltpu.VMEM((_PCHUNK // 4, _TPAD), jnp.float32),
        ],
        compiler_params=pltpu.CompilerParams(
            needs_layout_passes=False, use_tc_tiling_on_sc=False),
    )(p0, p1)


_PAR = 8


def _body(xt_hbm, proj_hbm, out_hbm, idx_v, *scratch):
    sbufs = scratch[:_PAR]
    bufs = scratch[_PAR:2 * _PAR]
    out_v = scratch[2 * _PAR]
    sems = scratch[2 * _PAR + 1:]

    wid = lax.axis_index("s") * _NC + lax.axis_index("c")
    b0 = wid * _BPW

    lanes = lax.iota(jnp.int32, 16)

    pltpu.sync_copy(xt_hbm.at[:, pl.ds(b0, _BPW)], idx_v)

    def shift(l):
        sidx = sbufs[l % _PAR]

        def sgrp(g, carry):
            sl = pl.ds(g * 16, 16)
            sidx[sl] = lax.shift_right_logical(idx_v[l, sl], 2)
            return carry

        lax.fori_loop(0, _GROUPS, sgrp, 0)

    def fire(l):
        buf = bufs[l % _PAR]
        sidx = sbufs[l % _PAR]
        sem = sems[l % _PAR]
        handles = []
        for j in range(_NSUB):
            handles.append(pltpu.async_copy(
                proj_hbm.at[sidx.at[pl.ds(j * _SUB, _SUB)]],
                buf.at[pl.ds(j * _SUB, _SUB), :],
                sem,
            ))
        return handles

    pending = {}
    for l in range(_PAR - 1):
        shift(l)
        pending[l] = fire(l)
    for l in range(HIST):
        for h in pending.pop(l):
            h.wait()
        if l + _PAR - 1 < HIST:
            shift(l + _PAR - 1)
            pending[l + _PAR - 1] = fire(l + _PAR - 1)
        rows = bufs[l % _PAR]

        def move(g, carry):
            row_ids = g * 16 + lanes
            sl = pl.ds(g * 16, 16)
            c0 = lax.shift_left(jnp.bitwise_and(idx_v[l, sl], 3), 1)
            out_v[l, 0, sl] = plsc.load_gather(rows, [row_ids, c0])
            out_v[l, 1, sl] = plsc.load_gather(rows, [row_ids, c0 + 1])
            return carry

        lax.fori_loop(0, _GROUPS, move, 0)

    pltpu.sync_copy(out_v, out_hbm.at[:, :, pl.ds(b0, _BPW)])


@jax.jit
def _run(xt, proj):
    mesh = plsc.VectorSubcoreMesh(core_axis_name="c", subcore_axis_name="s")
    return pl.kernel(
        _body,
        out_type=jax.ShapeDtypeStruct((HIST, TAG_SIZE, BATCH), jnp.float32),
        mesh=mesh,
        scratch_types=(
            [pltpu.VMEM((HIST, _BPW), jnp.int32)]
            + [pltpu.VMEM((_BPW,), jnp.int32) for _ in range(_PAR)]
            + [pltpu.VMEM((_BPW, _TPAD), jnp.float32) for _ in range(_PAR)]
            + [pltpu.VMEM((HIST, TAG_SIZE, _BPW), jnp.float32)]
            + [pltpu.SemaphoreType.DMA for _ in range(_PAR)]
        ),
        compiler_params=pltpu.CompilerParams(
            needs_layout_passes=False, use_tc_tiling_on_sc=False),
    )(xt, proj)


def kernel(x, emb, W, b):
    xt = jnp.swapaxes(x, 0, 1).astype(jnp.int32)
    embt = jnp.swapaxes(emb, 0, 1)
    b2d = b.astype(jnp.float32).reshape(TAG_SIZE, 1)
    p0, p1 = _tc_proj(embt, W.astype(jnp.float32), b2d)
    proj = _sc_pack(p0, p1)
    out_t = _run(xt, proj)
    return jnp.transpose(out_t, (2, 0, 1))
